# Initial kernel scaffold; baseline (speedup 1.0000x reference)
import jax
import jax.numpy as jnp
from jax.experimental import pallas as pl


def kernel(inputs, codebook):
    raise NotImplementedError("write your pallas kernel here")


# speedup vs baseline: 2.3525x; 2.3525x over previous
import jax
import jax.numpy as jnp
from jax.experimental import pallas as pl

_NUM_CODES = 8192
_DIM = 32
_TB = 256


def _vq_block(x_ref, cb_ref, out_ref):
    x = x_ref[...]
    cb = cb_ref[...]
    a = jnp.sum(x * x, axis=1, keepdims=True)
    b = jnp.sum(cb * cb, axis=0, keepdims=True)
    m = jnp.dot(x, cb, preferred_element_type=jnp.float32)
    norms = (a + b) - 2.0 * m
    vmin = jnp.min(norms, axis=1, keepdims=True)
    iota = jax.lax.broadcasted_iota(jnp.int32, norms.shape, 1)
    idx = jnp.min(jnp.where(norms == vmin, iota, _NUM_CODES), axis=1)
    onehot = (iota == idx[:, None]).astype(jnp.float32)
    q = jax.lax.dot_general(onehot, cb, (((1,), (1,)), ((), ())),
                            preferred_element_type=jnp.float32)
    out_ref[...] = x + (q - x)


def kernel(inputs, codebook):
    original_shape = inputs.shape
    x = inputs.reshape(-1, _DIM)
    n = x.shape[0]
    grid = (n // _TB,)
    out = pl.pallas_call(
        _vq_block,
        grid=grid,
        in_specs=[
            pl.BlockSpec((_TB, _DIM), lambda i: (i, 0)),
            pl.BlockSpec((_DIM, _NUM_CODES), lambda i: (0, 0)),
        ],
        out_specs=pl.BlockSpec((_TB, _DIM), lambda i: (i, 0)),
        out_shape=jax.ShapeDtypeStruct((n, _DIM), jnp.float32),
    )(x, codebook)
    return out.reshape(original_shape)
